# baseline (speedup 1.0000x reference)
import functools

import numpy as np
import jax
import jax.numpy as jnp
from jax import lax
from jax.experimental import pallas as pl
from jax.experimental.pallas import tpu as pltpu
from jax.experimental.pallas import tpu_sc as plsc

N_FEATURES = 26
INPUT_DIM = 101
OUT_DIM = 3
BATCH = 16384
ROW = N_FEATURES * OUT_DIM
NUM_WORKERS = 32
ROWS_PER_W = BATCH // NUM_WORKERS
BLOCK_ROWS = 8
BLOCK_ELEMS = BLOCK_ROWS * ROW
VECS_PER_BLOCK = BLOCK_ELEMS // 16
BLOCKS_PER_W = ROWS_PER_W // BLOCK_ROWS
IDX_PER_W = ROWS_PER_W * N_FEATURES
OUT_PER_W = ROWS_PER_W * ROW
TAB_SIZE = N_FEATURES * INPUT_DIM * OUT_DIM

_pos = np.arange(BLOCK_ELEMS)
_brow = _pos // ROW
_col = _pos % ROW
_feat = _col // OUT_DIM
_comp = _col % OUT_DIM
_TMPL = ((_brow * N_FEATURES + _feat)
         | ((_feat * (INPUT_DIM * OUT_DIM) + _comp) << 16)).astype(np.int32)


def _sc_embed(idx_flat, tab_flat, tmpl):
    mesh = plsc.VectorSubcoreMesh(core_axis_name="c", subcore_axis_name="s")

    @functools.partial(
        pl.kernel,
        mesh=mesh,
        out_type=jax.ShapeDtypeStruct((BATCH * ROW,), jnp.float32),
        compiler_params=pltpu.CompilerParams(
            needs_layout_passes=False,
            skip_device_barrier=True,
            disable_bounds_checks=True,
            disable_semaphore_checks=True,
        ),
        scratch_types=[
            pltpu.VMEM((IDX_PER_W,), jnp.int32),
            pltpu.VMEM((TAB_SIZE,), jnp.float32),
            pltpu.VMEM((BLOCK_ELEMS,), jnp.int32),
            pltpu.VMEM((OUT_PER_W,), jnp.float32),
        ],
    )
    def k(idx_hbm, tab_hbm, tp_hbm, out_hbm, idx_v, tab_v, tp_v, out_v):
        wid = lax.axis_index("s") * 2 + lax.axis_index("c")
        pltpu.sync_copy(idx_hbm.at[pl.ds(wid * IDX_PER_W, IDX_PER_W)], idx_v)
        pltpu.sync_copy(tab_hbm, tab_v)
        pltpu.sync_copy(tp_hbm, tp_v)

        @plsc.parallel_loop(0, BLOCKS_PER_W, unroll=2)
        def blk(b):
            ibase = b * (BLOCK_ROWS * N_FEATURES)
            obase = b * BLOCK_ELEMS
            for j in range(VECS_PER_BLOCK):
                p = tp_v[pl.ds(j * 16, 16)]
                a = p & 0xFFFF
                t = p >> 16
                iv = plsc.load_gather(idx_v, [a + ibase])
                e = plsc.load_gather(tab_v, [t + iv * OUT_DIM])
                out_v[pl.ds(obase + j * 16, 16)] = e

        pltpu.sync_copy(out_v, out_hbm.at[pl.ds(wid * OUT_PER_W, OUT_PER_W)])

    return k(idx_flat, tab_flat, tmpl)


def kernel(inputs, tables):
    out = _sc_embed(inputs.reshape(-1), tables.reshape(-1), _TMPL)
    return out.reshape(BATCH, ROW)


# speedup vs baseline: 1.5453x; 1.5453x over previous
import functools

import numpy as np
import jax
import jax.numpy as jnp
from jax import lax
from jax.experimental import pallas as pl
from jax.experimental.pallas import tpu as pltpu
from jax.experimental.pallas import tpu_sc as plsc

N_FEATURES = 26
INPUT_DIM = 101
OUT_DIM = 3
BATCH = 16384
ROW = N_FEATURES * OUT_DIM
NUM_WORKERS = 32
ROWS_PER_W = BATCH // NUM_WORKERS
CHUNK_ROWS = 64
NCHUNKS = ROWS_PER_W // CHUNK_ROWS
NGROUPS = 5
TAIL = ROW - 16 * (NGROUPS - 1)
TAB_SIZE = N_FEATURES * INPUT_DIM * OUT_DIM

_col = np.minimum(np.arange(NGROUPS * 16), ROW - 1)
_feat = (_col // OUT_DIM).astype(np.int32)
_tb = (_feat * (INPUT_DIM * OUT_DIM) + (_col % OUT_DIM)).astype(np.int32)
_feat[ROW:] = 0
_tb[ROW:] = 0
_CONSTS = np.concatenate([_feat, _tb])


def _sc_embed(idx, tab_flat, consts):
    mesh = plsc.VectorSubcoreMesh(core_axis_name="c", subcore_axis_name="s")

    @functools.partial(
        pl.kernel,
        mesh=mesh,
        out_type=jax.ShapeDtypeStruct((BATCH, ROW), jnp.float32),
        compiler_params=pltpu.CompilerParams(
            needs_layout_passes=False,
            disable_bounds_checks=True,
        ),
        scratch_types=[
            pltpu.VMEM((ROWS_PER_W, N_FEATURES), jnp.int32),
            pltpu.VMEM((TAB_SIZE,), jnp.float32),
            pltpu.VMEM((NGROUPS * 32,), jnp.int32),
            pltpu.VMEM((CHUNK_ROWS, ROW), jnp.float32),
        ],
    )
    def k(idx_hbm, tab_hbm, c_hbm, out_hbm, idx_v, tab_v, c_v, out_v):
        wid = lax.axis_index("s") * 2 + lax.axis_index("c")
        base = wid * ROWS_PER_W
        pltpu.sync_copy(idx_hbm.at[pl.ds(base, ROWS_PER_W)], idx_v)
        pltpu.sync_copy(tab_hbm, tab_v)
        pltpu.sync_copy(c_hbm, c_v)

        feat_g = [c_v[pl.ds(g * 16, 16)] for g in range(NGROUPS)]
        tb_g = [c_v[pl.ds(80 + g * 16, 16)] for g in range(NGROUPS)]
        lanes = lax.iota(jnp.int32, 16)
        tail_mask = lanes < TAIL
        tail_cols = lanes + (16 * (NGROUPS - 1))

        def chunk(c, _):
            rbase = c * CHUNK_ROWS

            @plsc.parallel_loop(0, CHUNK_ROWS, unroll=4)
            def row(r):
                rv = jnp.full((16,), rbase + r, dtype=jnp.int32)
                cv = jnp.full((16,), r, dtype=jnp.int32)
                for g in range(NGROUPS - 1):
                    iv = plsc.load_gather(idx_v, [rv, feat_g[g]])
                    e = plsc.load_gather(tab_v, [tb_g[g] + iv * OUT_DIM])
                    out_v[r, pl.ds(g * 16, 16)] = e
                g = NGROUPS - 1
                iv = plsc.load_gather(idx_v, [rv, feat_g[g]])
                e = plsc.load_gather(tab_v, [tb_g[g] + iv * OUT_DIM])
                plsc.store_scatter(out_v, [cv, tail_cols], e, mask=tail_mask)

            pltpu.sync_copy(out_v, out_hbm.at[pl.ds(base + rbase, CHUNK_ROWS)])
            return _

        lax.fori_loop(0, NCHUNKS, chunk, 0)

    return k(idx, tab_flat, consts)


def kernel(inputs, tables):
    return _sc_embed(inputs, tables.reshape(-1), _CONSTS)


# speedup vs baseline: 2.3643x; 1.5300x over previous
import functools

import jax
import jax.numpy as jnp
from jax import lax
from jax.experimental import pallas as pl
from jax.experimental.pallas import tpu as pltpu
from jax.experimental.pallas import tpu_sc as plsc

N_FEATURES = 26
INPUT_DIM = 101
OUT_DIM = 3
BATCH = 16384
ROW = N_FEATURES * OUT_DIM
NUM_WORKERS = 32
COLS_PER_W = BATCH // NUM_WORKERS
NVEC = COLS_PER_W // 16


def _sc_embed(idx_t, tab_t):
    mesh = plsc.VectorSubcoreMesh(core_axis_name="c", subcore_axis_name="s")

    @functools.partial(
        pl.kernel,
        mesh=mesh,
        out_type=jax.ShapeDtypeStruct((ROW, BATCH), jnp.float32),
        compiler_params=pltpu.CompilerParams(
            needs_layout_passes=False,
            disable_bounds_checks=True,
        ),
        scratch_types=[
            pltpu.VMEM((N_FEATURES, COLS_PER_W), jnp.int32),
            pltpu.VMEM((OUT_DIM, N_FEATURES, INPUT_DIM), jnp.float32),
            pltpu.VMEM((ROW, COLS_PER_W), jnp.float32),
        ],
    )
    def k(idx_hbm, tab_hbm, out_hbm, idx_v, tab_v, out_v):
        wid = lax.axis_index("s") * 2 + lax.axis_index("c")
        base = wid * COLS_PER_W
        pltpu.sync_copy(idx_hbm.at[:, pl.ds(base, COLS_PER_W)], idx_v)
        pltpu.sync_copy(tab_hbm, tab_v)

        @plsc.parallel_loop(0, NVEC, unroll=2)
        def vec(v):
            c0 = v * 16
            for f in range(N_FEATURES):
                iv = idx_v[f, pl.ds(c0, 16)]
                for d in range(OUT_DIM):
                    e = plsc.load_gather(tab_v.at[d, f], [iv])
                    out_v[OUT_DIM * f + d, pl.ds(c0, 16)] = e

        pltpu.sync_copy(out_v, out_hbm.at[:, pl.ds(base, COLS_PER_W)])

    return k(idx_t, tab_t)


def kernel(inputs, tables):
    out_t = _sc_embed(inputs.T, tables.transpose(2, 0, 1))
    return out_t.T
